# Initial kernel scaffold; baseline (speedup 1.0000x reference)
import jax
import jax.numpy as jnp
from jax.experimental import pallas as pl


def kernel(QuatsLog, CameraPosition, Landmarks, K, frame_id, point_id):
    raise NotImplementedError("write your pallas kernel here")


# speedup vs baseline: 13.7857x; 13.7857x over previous
import functools

import jax
import jax.numpy as jnp
from jax import lax
from jax.experimental import pallas as pl
from jax.experimental.pallas import tpu as pltpu
from jax.experimental.pallas import tpu_sc as plsc

_NC = 2
_NS = 16
_L = 16
_NW = _NC * _NS


def _pose_tc_body(q_ref, c_ref, k_ref, lm_ref, pose_ref, aux_ref, lmr_ref):
    logq = q_ref[:]
    n = jnp.maximum(jnp.sqrt(jnp.sum(logq * logq, axis=1, keepdims=True)), 1e-8)
    vec = logq * jnp.sin(n) / n
    w = jnp.cos(n)
    qn = jnp.maximum(
        jnp.sqrt(w * w + jnp.sum(vec * vec, axis=1, keepdims=True)), 1e-12)
    w = w / qn
    vec = vec / qn
    x = vec[:, 0:1]
    y = vec[:, 1:2]
    z = vec[:, 2:3]
    tx, ty, tz = 2.0 * x, 2.0 * y, 2.0 * z
    twx, twy, twz = tx * w, ty * w, tz * w
    txx, txy, txz = tx * x, ty * x, tz * x
    tyy, tyz = ty * y, tz * y
    tzz = tz * z
    one = jnp.ones_like(x)
    cp = c_ref[:]
    cols = [
        one - (tyy + tzz), txy - twz, txz + twy,
        txy + twz, one - (txx + tzz), tyz - twx,
        txz - twy, tyz + twx, one - (txx + tyy),
        cp[:, 0:1], cp[:, 1:2], cp[:, 2:3],
        jnp.zeros((q_ref.shape[0], 4), jnp.float32),
    ]
    pose = jnp.concatenate(cols, axis=1)
    pose_ref[:] = pose.astype(jnp.bfloat16).astype(jnp.float32)
    lmr_ref[:] = lm_ref[:].astype(jnp.bfloat16).astype(jnp.float32)
    km = k_ref[:]
    aux_ref[:] = jnp.concatenate(
        [
            jnp.broadcast_to(km[0:1, 0:1], (1, 128)),
            jnp.broadcast_to(km[1:2, 1:2], (1, 128)),
            jnp.broadcast_to(km[0:1, 2:3], (1, 128)),
            jnp.broadcast_to(km[1:2, 2:3], (1, 128)),
            jnp.zeros((4, 128), jnp.float32),
        ],
        axis=0,
    )


def _make_pose_tables(quats_log, camera_position, k_mat, landmarks):
    n_kf = quats_log.shape[0]
    return pl.pallas_call(
        _pose_tc_body,
        out_shape=(
            jax.ShapeDtypeStruct((n_kf, 16), jnp.float32),
            jax.ShapeDtypeStruct((8, 128), jnp.float32),
            jax.ShapeDtypeStruct(landmarks.shape, jnp.float32),
        ),
    )(quats_log, camera_position, k_mat, landmarks)


def _make_sc_project(m_obs, n_kf, n_mp):
    chunk = m_obs // _NW
    steps = chunk // _L
    mesh = plsc.VectorSubcoreMesh(core_axis_name="c", subcore_axis_name="s")

    @functools.partial(
        pl.kernel,
        out_type=jax.ShapeDtypeStruct((m_obs * 2,), jnp.float32),
        mesh=mesh,
        compiler_params=pltpu.CompilerParams(needs_layout_passes=False),
        scratch_types=[
            pltpu.VMEM((n_kf * 16,), jnp.float32),
            pltpu.VMEM((1024,), jnp.float32),
            pltpu.VMEM((n_mp * 3,), jnp.float32),
            pltpu.VMEM((chunk,), jnp.int32),
            pltpu.VMEM((chunk,), jnp.int32),
            pltpu.VMEM((chunk * 2,), jnp.float32),
        ],
    )
    def sc_project(pose_hbm, aux_hbm, lm_hbm, fid_hbm, pid_hbm, out_hbm,
                   pose_v, aux_v, lm_v, fid_v, pid_v, out_v):
        wid = lax.axis_index("s") * _NC + lax.axis_index("c")
        base = wid * chunk
        pltpu.sync_copy(pose_hbm, pose_v)
        pltpu.sync_copy(aux_hbm, aux_v)
        pltpu.sync_copy(lm_hbm, lm_v)
        pltpu.sync_copy(fid_hbm.at[pl.ds(base, chunk)], fid_v)
        pltpu.sync_copy(pid_hbm.at[pl.ds(base, chunk)], pid_v)

        fxv = aux_v[pl.ds(0 * 128, _L)]
        fyv = aux_v[pl.ds(1 * 128, _L)]
        cxv = aux_v[pl.ds(2 * 128, _L)]
        cyv = aux_v[pl.ds(3 * 128, _L)]
        lane = lax.iota(jnp.int32, _L)

        def step(i, carry):
            off = i * _L
            fid16 = fid_v[pl.ds(off, _L)] * 16
            pid3 = pid_v[pl.ds(off, _L)] * 3
            r00 = plsc.load_gather(pose_v, [fid16])
            r01 = plsc.load_gather(pose_v, [fid16 + 1])
            r02 = plsc.load_gather(pose_v, [fid16 + 2])
            r10 = plsc.load_gather(pose_v, [fid16 + 3])
            r11 = plsc.load_gather(pose_v, [fid16 + 4])
            r12 = plsc.load_gather(pose_v, [fid16 + 5])
            r20 = plsc.load_gather(pose_v, [fid16 + 6])
            r21 = plsc.load_gather(pose_v, [fid16 + 7])
            r22 = plsc.load_gather(pose_v, [fid16 + 8])
            tx = plsc.load_gather(pose_v, [fid16 + 9])
            ty = plsc.load_gather(pose_v, [fid16 + 10])
            tz = plsc.load_gather(pose_v, [fid16 + 11])
            px = plsc.load_gather(lm_v, [pid3])
            py = plsc.load_gather(lm_v, [pid3 + 1])
            pz = plsc.load_gather(lm_v, [pid3 + 2])
            xc = r00 * px + r01 * py + r02 * pz + tx
            yc = r10 * px + r11 * py + r12 * pz + ty
            zc = r20 * px + r21 * py + r22 * pz + tz
            s = jnp.where(jnp.abs(zc) > 1e-8, 1.0 / zc, jnp.ones_like(zc))
            u = (xc * s) * fxv + cxv
            v = (yc * s) * fyv + cyv
            rows2 = (off + lane) * 2
            plsc.store_scatter(out_v, [rows2], u)
            plsc.store_scatter(out_v, [rows2 + 1], v)
            return carry

        lax.fori_loop(0, steps, step, 0)
        pltpu.sync_copy(out_v, out_hbm.at[pl.ds(base * 2, chunk * 2)])

    return sc_project


def kernel(QuatsLog, CameraPosition, Landmarks, K, frame_id, point_id):
    pose, aux, lmr = _make_pose_tables(QuatsLog, CameraPosition, K, Landmarks)
    m_obs = frame_id.shape[0]
    fid = frame_id.reshape(m_obs)
    pid = point_id.reshape(m_obs)
    sc_project = _make_sc_project(m_obs, QuatsLog.shape[0], Landmarks.shape[0])
    out_flat = sc_project(pose.reshape(-1), aux.reshape(-1),
                          lmr.reshape(-1), fid, pid)
    return out_flat.reshape(m_obs, 2)
